# baseline (speedup 1.0000x reference)
import functools

import jax
import jax.numpy as jnp
from jax import lax
from jax.experimental import pallas as pl
from jax.experimental.pallas import tpu as pltpu
from jax.experimental.pallas import tpu_sc as plsc

_VOCAB = 100000
_EMBED_DIM = 64
_N_TOKENS = 10
_BATCH = 32
_SEQ = 2048

_CHUNK = 128
_N_CHUNKS = _SEQ // _CHUNK
_EG = _EMBED_DIM // 8


def _build_sc_kernel():
    mesh = plsc.VectorSubcoreMesh(core_axis_name="c", subcore_axis_name="s")

    @functools.partial(
        pl.kernel,
        mesh=mesh,
        compiler_params=pltpu.CompilerParams(
            use_tc_tiling_on_sc=False, needs_layout_passes=False
        ),
        out_type=jax.ShapeDtypeStruct(
            (_BATCH, _EG, _N_CHUNKS, 8, _CHUNK), jnp.float32
        ),
        scratch_types=[
            pltpu.VMEM((_N_CHUNKS, _CHUNK), jnp.int32),
            pltpu.VMEM((_CHUNK, _EMBED_DIM), jnp.float32),
            pltpu.VMEM((_CHUNK, _EMBED_DIM), jnp.float32),
            pltpu.VMEM((_EMBED_DIM, _CHUNK), jnp.float32),
            pltpu.VMEM((_EMBED_DIM, _CHUNK), jnp.float32),
            pltpu.VMEM((_N_TOKENS, _EMBED_DIM), jnp.float32),
            pltpu.SemaphoreType.DMA,
            pltpu.SemaphoreType.DMA,
            pltpu.SemaphoreType.DMA,
            pltpu.SemaphoreType.DMA,
        ],
    )
    def k(tok_hbm, table_hbm, learned_hbm, out_hbm,
          idx_v, rows0, rows1, tp0, tp1, learned_v, gsem0, gsem1, wsem0, wsem1):
        wid = lax.axis_index("s") * 2 + lax.axis_index("c")

        pltpu.sync_copy(tok_hbm.at[wid], idx_v)
        pltpu.sync_copy(learned_hbm, learned_v)

        bufs = (rows0, rows1)
        tbufs = (tp0, tp1)
        gsems = (gsem0, gsem1)
        wsems = (wsem0, wsem1)
        gcopies = [None, None]
        wcopies = [[], []]

        iota = lax.iota(jnp.int32, 16)

        def transpose_chunk(buf, tbuf):
            @pl.loop(0, _EMBED_DIM)
            def _(e):
                e_splat = jnp.full((16,), e, jnp.int32)
                for t0 in range(_CHUNK // 16):
                    vals = plsc.load_gather(buf, [iota + (t0 * 16), e_splat])
                    tbuf[e, pl.ds(t0 * 16, 16)] = vals

        gcopies[0] = pltpu.async_copy(table_hbm.at[idx_v.at[0]], bufs[0], gsems[0])
        for j in range(_N_CHUNKS):
            p = j % 2
            gcopies[p].wait()
            if j + 1 < _N_CHUNKS:
                gcopies[1 - p] = pltpu.async_copy(
                    table_hbm.at[idx_v.at[j + 1]], bufs[1 - p], gsems[1 - p]
                )
            if j == 0:
                for r in range(_N_TOKENS):
                    for c in range(_EMBED_DIM // 16):
                        bufs[p][r, pl.ds(c * 16, 16)] = learned_v[r, pl.ds(c * 16, 16)]
            for cp in wcopies[p]:
                cp.wait()
            wcopies[p] = []
            transpose_chunk(bufs[p], tbufs[p])
            for g in range(_EG):
                wcopies[p].append(
                    pltpu.async_copy(
                        tbufs[p].at[pl.ds(g * 8, 8)],
                        out_hbm.at[wid, g, j],
                        wsems[p],
                    )
                )
        for p in (0, 1):
            for cp in wcopies[p]:
                cp.wait()

    return k


_sc_kernel = _build_sc_kernel()


@jax.jit
def kernel(tokens, wte_weight, learned_embedding):
    tok = tokens.astype(jnp.int32).reshape(_BATCH, _N_CHUNKS, _CHUNK)
    out = _sc_kernel(tok, wte_weight, learned_embedding)
    return out.transpose(0, 2, 4, 1, 3).reshape(_BATCH, _SEQ, _EMBED_DIM)


# speedup vs baseline: 1.2541x; 1.2541x over previous
import functools

import jax
import jax.numpy as jnp
from jax import lax
from jax.experimental import pallas as pl
from jax.experimental.pallas import tpu as pltpu
from jax.experimental.pallas import tpu_sc as plsc

_VOCAB = 100000
_EMBED_DIM = 64
_N_TOKENS = 10
_BATCH = 32
_SEQ = 2048

_CHUNK = 128
_N_CHUNKS = _SEQ // _CHUNK
_EG = _EMBED_DIM // 8


def _build_sc_kernel():
    mesh = plsc.VectorSubcoreMesh(core_axis_name="c", subcore_axis_name="s")

    @functools.partial(
        pl.kernel,
        mesh=mesh,
        compiler_params=pltpu.CompilerParams(
            use_tc_tiling_on_sc=False, needs_layout_passes=False
        ),
        out_type=jax.ShapeDtypeStruct(
            (_BATCH, _EG, _N_CHUNKS, 8, _CHUNK), jnp.float32
        ),
        scratch_types=[
            pltpu.VMEM((_N_CHUNKS, _CHUNK), jnp.int32),
            pltpu.VMEM((_CHUNK, _EMBED_DIM), jnp.float32),
            pltpu.VMEM((_CHUNK, _EMBED_DIM), jnp.float32),
            pltpu.VMEM((_EMBED_DIM, _CHUNK), jnp.float32),
            pltpu.VMEM((_EMBED_DIM, _CHUNK), jnp.float32),
            pltpu.VMEM((_N_TOKENS, _EMBED_DIM), jnp.float32),
            pltpu.SemaphoreType.DMA,
            pltpu.SemaphoreType.DMA,
            pltpu.SemaphoreType.DMA,
            pltpu.SemaphoreType.DMA,
        ],
    )
    def k(tok_hbm, table_hbm, learned_hbm, out_hbm,
          idx_v, rows0, rows1, tp0, tp1, learned_v, gsem0, gsem1, wsem0, wsem1):
        wid = lax.axis_index("s") * 2 + lax.axis_index("c")

        pltpu.sync_copy(tok_hbm.at[wid], idx_v)
        pltpu.sync_copy(learned_hbm, learned_v)

        bufs = (rows0, rows1)
        tbufs = (tp0, tp1)
        gsems = (gsem0, gsem1)
        wsems = (wsem0, wsem1)
        gcopies = [None, None]
        wcopies = [[], []]

        iota = lax.iota(jnp.int32, 16)
        rows16 = [iota + t0 * 16 for t0 in range(_CHUNK // 16)]

        def transpose_chunk(buf, tbuf):
            @plsc.parallel_loop(0, _EMBED_DIM, 1, unroll=8)
            def _(e):
                e_splat = jnp.full((16,), e, jnp.int32)
                for t0 in range(_CHUNK // 16):
                    vals = plsc.load_gather(buf, [rows16[t0], e_splat])
                    tbuf[e, pl.ds(t0 * 16, 16)] = vals

        gcopies[0] = pltpu.async_copy(table_hbm.at[idx_v.at[0]], bufs[0], gsems[0])
        for j in range(_N_CHUNKS):
            p = j % 2
            gcopies[p].wait()
            if j + 1 < _N_CHUNKS:
                gcopies[1 - p] = pltpu.async_copy(
                    table_hbm.at[idx_v.at[j + 1]], bufs[1 - p], gsems[1 - p]
                )
            if j == 0:
                for r in range(_N_TOKENS):
                    for c in range(_EMBED_DIM // 16):
                        bufs[p][r, pl.ds(c * 16, 16)] = learned_v[r, pl.ds(c * 16, 16)]
            for cp in wcopies[p]:
                cp.wait()
            wcopies[p] = []
            transpose_chunk(bufs[p], tbufs[p])
            for g in range(_EG):
                wcopies[p].append(
                    pltpu.async_copy(
                        tbufs[p].at[pl.ds(g * 8, 8)],
                        out_hbm.at[wid, g, j],
                        wsems[p],
                    )
                )
        for p in (0, 1):
            for cp in wcopies[p]:
                cp.wait()

    return k


_sc_kernel = _build_sc_kernel()


@jax.jit
def kernel(tokens, wte_weight, learned_embedding):
    tok = tokens.astype(jnp.int32).reshape(_BATCH, _N_CHUNKS, _CHUNK)
    out = _sc_kernel(tok, wte_weight, learned_embedding)
    return out.transpose(0, 2, 4, 1, 3).reshape(_BATCH, _SEQ, _EMBED_DIM)


# speedup vs baseline: 1.2690x; 1.0119x over previous
import functools

import jax
import jax.numpy as jnp
from jax import lax
from jax.experimental import pallas as pl
from jax.experimental.pallas import tpu as pltpu
from jax.experimental.pallas import tpu_sc as plsc

_VOCAB = 100000
_EMBED_DIM = 64
_N_TOKENS = 10
_BATCH = 32
_SEQ = 2048

_CHUNK = 128
_N_CHUNKS = _SEQ // _CHUNK
_EG = _EMBED_DIM // 8


def _build_sc_kernel():
    mesh = plsc.VectorSubcoreMesh(core_axis_name="c", subcore_axis_name="s")

    @functools.partial(
        pl.kernel,
        mesh=mesh,
        compiler_params=pltpu.CompilerParams(
            use_tc_tiling_on_sc=False, needs_layout_passes=False
        ),
        out_type=jax.ShapeDtypeStruct(
            (_BATCH, _EG, _N_CHUNKS, 8, _CHUNK), jnp.float32
        ),
        scratch_types=[
            pltpu.VMEM((_N_CHUNKS, _CHUNK), jnp.int32),
            pltpu.VMEM((_CHUNK, _EMBED_DIM), jnp.float32),
            pltpu.VMEM((_CHUNK, _EMBED_DIM), jnp.float32),
            pltpu.VMEM((_EMBED_DIM, _CHUNK), jnp.float32),
            pltpu.VMEM((_EMBED_DIM, _CHUNK), jnp.float32),
            pltpu.VMEM((_N_TOKENS, _EMBED_DIM), jnp.float32),
            pltpu.SemaphoreType.DMA,
            pltpu.SemaphoreType.DMA,
            pltpu.SemaphoreType.DMA,
            pltpu.SemaphoreType.DMA,
        ],
    )
    def k(tok_hbm, table_hbm, learned_hbm, out_hbm,
          idx_v, rows0, rows1, tp0, tp1, learned_v, gsem0, gsem1, wsem0, wsem1):
        wid = lax.axis_index("s") * 2 + lax.axis_index("c")

        pltpu.sync_copy(tok_hbm.at[wid], idx_v)
        pltpu.sync_copy(learned_hbm, learned_v)

        bufs = (rows0, rows1)
        tbufs = (tp0, tp1)
        gsems = (gsem0, gsem1)
        wsems = (wsem0, wsem1)
        gcopies = [None, None]
        wcopies = [[], []]

        iota = lax.iota(jnp.int32, 16)
        rows16 = [iota + t0 * 16 for t0 in range(_CHUNK // 16)]

        def transpose_chunk(buf, tbuf):
            for t0 in range(_CHUNK // 16):
                rows = rows16[t0]

                @plsc.parallel_loop(0, _EMBED_DIM, 1, unroll=4)
                def _(e, rows=rows, t0=t0):
                    e_splat = jnp.full((16,), e, jnp.int32)
                    vals = plsc.load_gather(buf, [rows, e_splat])
                    tbuf[e, pl.ds(t0 * 16, 16)] = vals

        gcopies[0] = pltpu.async_copy(table_hbm.at[idx_v.at[0]], bufs[0], gsems[0])
        for j in range(_N_CHUNKS):
            p = j % 2
            gcopies[p].wait()
            if j + 1 < _N_CHUNKS:
                gcopies[1 - p] = pltpu.async_copy(
                    table_hbm.at[idx_v.at[j + 1]], bufs[1 - p], gsems[1 - p]
                )
            if j == 0:
                for r in range(_N_TOKENS):
                    for c in range(_EMBED_DIM // 16):
                        bufs[p][r, pl.ds(c * 16, 16)] = learned_v[r, pl.ds(c * 16, 16)]
            for cp in wcopies[p]:
                cp.wait()
            wcopies[p] = []
            transpose_chunk(bufs[p], tbufs[p])
            for g in range(_EG):
                wcopies[p].append(
                    pltpu.async_copy(
                        tbufs[p].at[pl.ds(g * 8, 8)],
                        out_hbm.at[wid, g, j],
                        wsems[p],
                    )
                )
        for p in (0, 1):
            for cp in wcopies[p]:
                cp.wait()

    return k


_sc_kernel = _build_sc_kernel()


@jax.jit
def kernel(tokens, wte_weight, learned_embedding):
    tok = tokens.astype(jnp.int32).reshape(_BATCH, _N_CHUNKS, _CHUNK)
    out = _sc_kernel(tok, wte_weight, learned_embedding)
    return out.transpose(0, 2, 4, 1, 3).reshape(_BATCH, _SEQ, _EMBED_DIM)


# speedup vs baseline: 1.3200x; 1.0402x over previous
import functools

import jax
import jax.numpy as jnp
from jax import lax
from jax.experimental import pallas as pl
from jax.experimental.pallas import tpu as pltpu
from jax.experimental.pallas import tpu_sc as plsc

_VOCAB = 100000
_EMBED_DIM = 64
_N_TOKENS = 10
_BATCH = 32
_SEQ = 2048

_CHUNK = 128
_N_CHUNKS = _SEQ // _CHUNK
_EG = _EMBED_DIM // 8


def _build_sc_kernel():
    mesh = plsc.VectorSubcoreMesh(core_axis_name="c", subcore_axis_name="s")

    @functools.partial(
        pl.kernel,
        mesh=mesh,
        compiler_params=pltpu.CompilerParams(
            use_tc_tiling_on_sc=False, needs_layout_passes=False
        ),
        out_type=jax.ShapeDtypeStruct(
            (_BATCH, _EG, _N_CHUNKS, 8, _CHUNK), jnp.float32
        ),
        scratch_types=[
            pltpu.VMEM((_N_CHUNKS, _CHUNK), jnp.int32),
            pltpu.VMEM((_CHUNK, _EMBED_DIM), jnp.float32),
            pltpu.VMEM((_CHUNK, _EMBED_DIM), jnp.float32),
            pltpu.VMEM((_EMBED_DIM, _CHUNK), jnp.float32),
            pltpu.VMEM((_EMBED_DIM, _CHUNK), jnp.float32),
            pltpu.VMEM((_N_TOKENS, _EMBED_DIM), jnp.float32),
            pltpu.SemaphoreType.DMA,
            pltpu.SemaphoreType.DMA,
            pltpu.SemaphoreType.DMA,
            pltpu.SemaphoreType.DMA,
        ],
    )
    def k(tok_hbm, table_hbm, learned_hbm, out_hbm,
          idx_v, rows0, rows1, tp0, tp1, learned_v, gsem0, gsem1, wsem0, wsem1):
        wid = lax.axis_index("s") * 2 + lax.axis_index("c")

        pltpu.sync_copy(tok_hbm.at[wid], idx_v)
        pltpu.sync_copy(learned_hbm, learned_v)

        bufs = (rows0, rows1)
        tbufs = (tp0, tp1)
        gsems = (gsem0, gsem1)
        wsems = (wsem0, wsem1)
        gcopies = [None, None]
        wcopies = [[], []]

        iota = lax.iota(jnp.int32, 16)
        rows16 = [iota + t0 * 16 for t0 in range(_CHUNK // 16)]

        def transpose_chunk(buf, tbuf):
            @plsc.parallel_loop(0, _EMBED_DIM, 1, unroll=4)
            def _(e):
                e_splat = jnp.full((16,), e, jnp.int32)
                for t0 in range(_CHUNK // 16):
                    vals = plsc.load_gather(buf, [rows16[t0], e_splat])
                    tbuf[e, pl.ds(t0 * 16, 16)] = vals

        gcopies[0] = pltpu.async_copy(table_hbm.at[idx_v.at[0]], bufs[0], gsems[0])
        for j in range(_N_CHUNKS):
            p = j % 2
            gcopies[p].wait()
            if j + 1 < _N_CHUNKS:
                gcopies[1 - p] = pltpu.async_copy(
                    table_hbm.at[idx_v.at[j + 1]], bufs[1 - p], gsems[1 - p]
                )
            if j == 0:
                for r in range(_N_TOKENS):
                    for c in range(_EMBED_DIM // 16):
                        bufs[p][r, pl.ds(c * 16, 16)] = learned_v[r, pl.ds(c * 16, 16)]
            for cp in wcopies[p]:
                cp.wait()
            wcopies[p] = []
            transpose_chunk(bufs[p], tbufs[p])
            for g in range(_EG):
                wcopies[p].append(
                    pltpu.async_copy(
                        tbufs[p].at[pl.ds(g * 8, 8)],
                        out_hbm.at[wid, g, j],
                        wsems[p],
                    )
                )
        for p in (0, 1):
            for cp in wcopies[p]:
                cp.wait()

    return k


_sc_kernel = _build_sc_kernel()


@jax.jit
def kernel(tokens, wte_weight, learned_embedding):
    tok = tokens.astype(jnp.int32).reshape(_BATCH, _N_CHUNKS, _CHUNK)
    out = _sc_kernel(tok, wte_weight, learned_embedding)
    return out.transpose(0, 2, 4, 1, 3).reshape(_BATCH, _SEQ, _EMBED_DIM)
